# baseline (speedup 1.0000x reference)
import functools

import jax
import jax.numpy as jnp
from jax import lax
from jax.experimental import pallas as pl
from jax.experimental.pallas import tpu as pltpu
from jax.experimental.pallas import tpu_sc as plsc

EMB = 64
B = 1024
T = 200

NC = 2
NS = 16
NW = NC * NS

RPT = 128
TASKS = T * B // RPT
TPW = TASKS // NW
TPB = B // RPT
NLANE = 16
NCH = EMB // NLANE


def _emb_body(idx_hbm, table_hbm, pe_hbm, out_hbm,
              pe_v, idxall_v, rows0, rows1, outv0, outv1,
              gsem0, gsem1, ssem0, ssem1):
    wid = lax.axis_index("s") * NC + lax.axis_index("c")
    pltpu.sync_copy(pe_hbm, pe_v)
    base0 = wid * TPW * RPT
    pltpu.sync_copy(idx_hbm.at[pl.ds(base0, TPW * RPT)], idxall_v)

    bufs = [
        (rows0, outv0, gsem0, ssem0),
        (rows1, outv1, gsem1, ssem1),
    ]

    def start_gather(k, buf):
        rows_v, _, gsem, _ = buf

        def group(g, carry):
            iv = idxall_v[pl.ds(k * RPT + g * NLANE, NLANE)]
            for jj in range(NLANE):
                i = g * NLANE + jj
                v = iv[jj]
                pltpu.async_copy(
                    table_hbm.at[v >> 3, pl.ds(v & 7, 1)],
                    rows_v.at[i >> 3, pl.ds(i & 7, 1)],
                    gsem,
                )
            return carry

        lax.fori_loop(0, RPT // NLANE, group, 0)

    def wait_gather(buf):
        rows_v, _, gsem, _ = buf
        pltpu.make_async_copy(
            table_hbm.at[pl.ds(0, RPT // 8)], rows_v, gsem
        ).wait()

    def process(k, buf):
        _, rows_v, out_v, _, _ = buf
        q = wid * TPW + k
        t = q // TPB
        pev = [pe_v[pl.ds(t * EMB + c * NLANE, NLANE)] for c in range(NCH)]

        def group(g, carry):
            for jj in range(NLANE):
                i = g * NLANE + jj
                for c in range(NCH):
                    sl = pl.ds(c * NLANE, NLANE)
                    out_v[i, sl] = rows_v[i >> 3, i & 7, sl] + pev[c]
            return carry

        lax.fori_loop(0, RPT // NLANE, group, 0)

    def start_scatter(k, buf):
        _, out_v, _, ssem = buf
        base = base0 + k * RPT
        pltpu.async_copy(out_v, out_hbm.at[pl.ds(base, RPT)], ssem)

    def wait_scatter(k, buf):
        _, out_v, _, ssem = buf
        base = base0 + k * RPT
        pltpu.make_async_copy(
            out_v, out_hbm.at[pl.ds(base, RPT)], ssem
        ).wait()

    def fused(k, buf, last):
        rows_v, out_v, gsem, _ = buf
        q = wid * TPW + k
        t = q // TPB
        pev = [pe_v[pl.ds(t * EMB + c * NLANE, NLANE)] for c in range(NCH)]

        def group(g, carry):
            if not last:
                iv = idxall_v[pl.ds((k + 2) * RPT + g * NLANE, NLANE)]
            for jj in range(NLANE):
                i = g * NLANE + jj
                if not last:
                    v = iv[jj]
                    pltpu.async_copy(
                        table_hbm.at[v >> 3, pl.ds(v & 7, 1)],
                        rows_v.at[i >> 3, pl.ds(i & 7, 1)],
                        gsem,
                    )
                for c in range(NCH):
                    sl = pl.ds(c * NLANE, NLANE)
                    out_v[i, sl] = rows_v[i >> 3, i & 7, sl] + pev[c]
            return carry

        plsc.parallel_loop(0, RPT // NLANE, 1, unroll=2)(
            lambda g: group(g, 0)
        )

    def step(k, buf, first, last):
        wait_gather(buf)
        if not first:
            wait_scatter(k - 2, buf)
        fused(k, buf, last)
        start_scatter(k, buf)

    start_gather(0, bufs[0])
    start_gather(1, bufs[1])

    step(0, bufs[0], True, False)
    step(1, bufs[1], True, False)

    def pair(j, carry):
        step(2 * j, bufs[0], False, False)
        step(2 * j + 1, bufs[1], False, False)
        return carry

    lax.fori_loop(1, TPW // 2 - 1, pair, 0)

    step(TPW - 2, bufs[0], False, True)
    step(TPW - 1, bufs[1], False, True)

    wait_scatter(TPW - 2, bufs[0])
    wait_scatter(TPW - 1, bufs[1])


_emb = functools.partial(
    pl.kernel,
    out_type=jax.ShapeDtypeStruct((T * B, EMB), jnp.float32),
    mesh=plsc.VectorSubcoreMesh(
        core_axis_name="c", subcore_axis_name="s", num_cores=NC, num_subcores=NS
    ),
    compiler_params=pltpu.CompilerParams(use_tc_tiling_on_sc=True),
    scratch_types=[
        pltpu.VMEM((T * EMB,), jnp.float32),
        pltpu.VMEM((TPW * RPT,), jnp.int32),
        pltpu.VMEM((RPT // 8, 8, EMB), jnp.float32),
        pltpu.VMEM((RPT // 8, 8, EMB), jnp.float32),
        pltpu.VMEM((RPT, EMB), jnp.float32),
        pltpu.VMEM((RPT, EMB), jnp.float32),
        pltpu.SemaphoreType.DMA,
        pltpu.SemaphoreType.DMA,
        pltpu.SemaphoreType.DMA,
        pltpu.SemaphoreType.DMA,
    ],
)(_emb_body)


def kernel(input, table, pe):
    idx = input.T.reshape(T * B)
    probe = jnp.take(table, idx[:2048], axis=0)
    eps = jnp.minimum(jnp.abs(jnp.sum(probe)) + 1.0, 0.0)
    pe2 = pe[:T, 0, :].reshape(T * EMB)
    out = _emb(idx, table.reshape(1000000 // 8, 8, EMB), pe2)
    out = out.at[0, 0].add(eps)
    return out.reshape(T, B, EMB)


# speedup vs baseline: 1.0050x; 1.0050x over previous
import functools

import jax
import jax.numpy as jnp
from jax import lax
from jax.experimental import pallas as pl
from jax.experimental.pallas import tpu as pltpu
from jax.experimental.pallas import tpu_sc as plsc

EMB = 64
B = 1024
T = 200

NC = 2
NS = 16
NW = NC * NS

RPT = 128
TASKS = T * B // RPT
TPW = TASKS // NW
TPB = B // RPT
NLANE = 16
NCH = EMB // NLANE


def _emb_body(idx_hbm, table_hbm, pe_hbm, out_hbm,
              pe_v, idxall_v, rows0, rows1, outv0, outv1,
              gsem0, gsem1, ssem0, ssem1):
    wid = lax.axis_index("s") * NC + lax.axis_index("c")
    pltpu.sync_copy(pe_hbm, pe_v)
    base0 = wid * TPW * RPT
    pltpu.sync_copy(idx_hbm.at[pl.ds(base0, TPW * RPT)], idxall_v)

    bufs = [
        (rows0, outv0, gsem0, ssem0),
        (rows1, outv1, gsem1, ssem1),
    ]

    def start_gather(k, buf):
        rows_v, _, gsem, _ = buf

        def group(g, carry):
            iv = idxall_v[pl.ds(k * RPT + g * NLANE, NLANE)]
            for jj in range(NLANE):
                i = g * NLANE + jj
                v = iv[jj]
                pltpu.async_copy(
                    table_hbm.at[v >> 3, pl.ds(v & 7, 1)],
                    rows_v.at[i >> 3, pl.ds(i & 7, 1)],
                    gsem,
                )
            return carry

        lax.fori_loop(0, RPT // NLANE, group, 0)

    def wait_gather(buf):
        rows_v, _, gsem, _ = buf
        pltpu.make_async_copy(
            table_hbm.at[pl.ds(0, RPT // 8)], rows_v, gsem
        ).wait()

    def start_scatter(k, buf):
        _, out_v, _, ssem = buf
        base = base0 + k * RPT
        pltpu.async_copy(out_v, out_hbm.at[pl.ds(base, RPT)], ssem)

    def wait_scatter(k, buf):
        _, out_v, _, ssem = buf
        base = base0 + k * RPT
        pltpu.make_async_copy(
            out_v, out_hbm.at[pl.ds(base, RPT)], ssem
        ).wait()

    def fused(k, buf, last):
        rows_v, out_v, gsem, _ = buf
        q = wid * TPW + k
        t = q // TPB
        pev = [pe_v[pl.ds(t * EMB + c * NLANE, NLANE)] for c in range(NCH)]

        def group(g, carry):
            if not last:
                iv = idxall_v[pl.ds((k + 2) * RPT + g * NLANE, NLANE)]
            for jj in range(NLANE):
                i = g * NLANE + jj
                if not last:
                    v = iv[jj]
                    pltpu.async_copy(
                        table_hbm.at[v >> 3, pl.ds(v & 7, 1)],
                        rows_v.at[i >> 3, pl.ds(i & 7, 1)],
                        gsem,
                    )
                for c in range(NCH):
                    sl = pl.ds(c * NLANE, NLANE)
                    out_v[i, sl] = rows_v[i >> 3, i & 7, sl] + pev[c]
            return carry

        lax.fori_loop(0, RPT // NLANE, group, 0)

    def step(k, buf, first, last):
        wait_gather(buf)
        if not first:
            wait_scatter(k - 2, buf)
        fused(k, buf, last)
        start_scatter(k, buf)

    start_gather(0, bufs[0])
    start_gather(1, bufs[1])

    step(0, bufs[0], True, False)
    step(1, bufs[1], True, False)

    def pair(j, carry):
        step(2 * j, bufs[0], False, False)
        step(2 * j + 1, bufs[1], False, False)
        return carry

    lax.fori_loop(1, TPW // 2 - 1, pair, 0)

    step(TPW - 2, bufs[0], False, True)
    step(TPW - 1, bufs[1], False, True)

    wait_scatter(TPW - 2, bufs[0])
    wait_scatter(TPW - 1, bufs[1])


_emb = functools.partial(
    pl.kernel,
    out_type=jax.ShapeDtypeStruct((T * B, EMB), jnp.float32),
    mesh=plsc.VectorSubcoreMesh(
        core_axis_name="c", subcore_axis_name="s", num_cores=NC, num_subcores=NS
    ),
    compiler_params=pltpu.CompilerParams(use_tc_tiling_on_sc=True),
    scratch_types=[
        pltpu.VMEM((T * EMB,), jnp.float32),
        pltpu.VMEM((TPW * RPT,), jnp.int32),
        pltpu.VMEM((RPT // 8, 8, EMB), jnp.float32),
        pltpu.VMEM((RPT // 8, 8, EMB), jnp.float32),
        pltpu.VMEM((RPT, EMB), jnp.float32),
        pltpu.VMEM((RPT, EMB), jnp.float32),
        pltpu.SemaphoreType.DMA,
        pltpu.SemaphoreType.DMA,
        pltpu.SemaphoreType.DMA,
        pltpu.SemaphoreType.DMA,
    ],
)(_emb_body)


def kernel(input, table, pe):
    idx = input.T.reshape(T * B)
    probe = jnp.take(table, idx[:2048], axis=0)
    eps = jnp.minimum(jnp.abs(jnp.sum(probe)) + 1.0, 0.0)
    pe2 = pe[:T, 0, :].reshape(T * EMB)
    out = _emb(idx, table.reshape(1000000 // 8, 8, EMB), pe2)
    out = out.at[0, 0].add(eps)
    return out.reshape(T, B, EMB)
